# baseline (speedup 1.0000x reference)
import jax
import jax.numpy as jnp
from jax import lax
from jax.experimental import pallas as pl
from jax.experimental.pallas import tpu as pltpu
from jax.experimental.pallas import tpu_sc as plsc

N = 10000
D = 128
HD = D // 2
E = 320000

NPAD = 10240
NC, NS = 2, 16
NW = NC * NS
CK = 128
EPAD = NW * CK * ((E + NW * CK - 1) // (NW * CK))
CH = EPAD // (NW * CK)
RPS = NPAD // NS

_mesh = plsc.VectorSubcoreMesh(core_axis_name="c", subcore_axis_name="s")


def _deg_body(dst_hbm, zeros_hbm, ones_hbm, out_hbm, dst_v, ones_b, fbuf,
              rbuf, deg_sh):
    cid = lax.axis_index("c")
    sid = lax.axis_index("s")
    wid = cid * NS + sid
    pltpu.sync_copy(zeros_hbm.at[pl.ds(sid * RPS, RPS)],
                    deg_sh.at[pl.ds(sid * RPS, RPS)])
    pltpu.sync_copy(ones_hbm, ones_b)
    pltpu.sync_copy(dst_hbm.at[wid], dst_v)
    plsc.subcore_barrier()

    @pl.loop(0, CH)
    def _(j):
        pltpu.sync_copy(ones_b, deg_sh.at[dst_v.at[j]], add=True)

    plsc.subcore_barrier()

    @pl.loop(0, RPS // CK)
    def _(c):
        base = sid * RPS + c * CK
        pltpu.sync_copy(deg_sh.at[pl.ds(base, CK)], rbuf)

        @pl.loop(0, CK)
        def _(rw):
            for gidx in range(HD // 16):
                v = rbuf[rw, pl.ds(gidx * 32, 32)]
                fbuf[rw, pl.ds(gidx * 16, 16)] = plsc.bitcast(v, jnp.float32)

        pltpu.sync_copy(fbuf, out_hbm.at[cid, pl.ds(base, CK)])


_sc_deg = pl.kernel(
    _deg_body,
    out_type=jax.ShapeDtypeStruct((NC, NPAD, HD), jnp.float32),
    mesh=_mesh,
    scratch_types=[
        pltpu.VMEM((CH, CK), jnp.int32),
        pltpu.VMEM((CK, D), jnp.bfloat16),
        pltpu.VMEM((CK, HD), jnp.float32),
        pltpu.VMEM((CK, D), jnp.bfloat16),
        pltpu.VMEM_SHARED((NPAD, D), jnp.bfloat16),
    ],
    compiler_params=pltpu.CompilerParams(use_tc_tiling_on_sc=False,
                                         needs_layout_passes=False),
)


NB = 4
OFF = 3


def _retype_f32_to_bf16(fbuf, bbuf):
    @pl.loop(0, CK)
    def _(rw):
        for gidx in range(HD // 16):
            v = fbuf[rw, pl.ds(gidx * 16, 16)]
            bbuf[rw, pl.ds(gidx * 32, 32)] = plsc.bitcast(v, jnp.bfloat16)


def _agg_body(g_hbm, src_hbm, dst_hbm, zeros_hbm, out_hbm,
              src_v, dst_v, f0, bbuf, rbuf, s_sh):
    cid = lax.axis_index("c")
    sid = lax.axis_index("s")
    wid = cid * NS + sid
    pltpu.sync_copy(src_hbm.at[wid], src_v)
    pltpu.sync_copy(dst_hbm.at[wid], dst_v)
    pltpu.sync_copy(zeros_hbm.at[pl.ds(sid * RPS, RPS)],
                    s_sh.at[pl.ds(sid * RPS, RPS)])
    plsc.subcore_barrier()

    @pl.loop(0, CH)
    def _(j):
        pltpu.sync_copy(g_hbm.at[src_v.at[j]], f0)
        _retype_f32_to_bf16(f0, bbuf)
        pltpu.sync_copy(bbuf, s_sh.at[dst_v.at[j]], add=True)

    plsc.subcore_barrier()

    @pl.loop(0, RPS // CK)
    def _(c):
        base = sid * RPS + c * CK
        pltpu.sync_copy(s_sh.at[pl.ds(base, CK)], rbuf)

        @pl.loop(0, CK)
        def _(rw):
            for gidx in range(HD // 16):
                v = rbuf[rw, pl.ds(gidx * 32, 32)]
                f0[rw, pl.ds(gidx * 16, 16)] = plsc.bitcast(v, jnp.float32)

        pltpu.sync_copy(f0, out_hbm.at[cid, pl.ds(base, CK)])


_sc_agg = pl.kernel(
    _agg_body,
    out_type=jax.ShapeDtypeStruct((NC, NPAD, HD), jnp.float32),
    mesh=_mesh,
    scratch_types=[
        pltpu.VMEM((CH, CK), jnp.int32),
        pltpu.VMEM((CH, CK), jnp.int32),
        pltpu.VMEM((CK, HD), jnp.float32),
        pltpu.VMEM((CK, D), jnp.bfloat16),
        pltpu.VMEM((CK, D), jnp.bfloat16),
        pltpu.VMEM_SHARED((NPAD, D), jnp.bfloat16),
    ],
    compiler_params=pltpu.CompilerParams(use_tc_tiling_on_sc=False,
                                         needs_layout_passes=False),
)


def _h_body(x_ref, w_ref, h_ref):
    h_ref[...] = jnp.dot(x_ref[...], w_ref[...],
                         preferred_element_type=jnp.float32,
                         precision=lax.Precision.HIGHEST)


_BLK1 = 1024


def _tc_h(x_pad, W):
    return pl.pallas_call(
        _h_body,
        grid=(NPAD // _BLK1,),
        in_specs=[
            pl.BlockSpec((_BLK1, D), lambda i: (i, 0)),
            pl.BlockSpec((D, D), lambda i: (0, 0)),
        ],
        out_specs=pl.BlockSpec((_BLK1, D), lambda i: (i, 0)),
        out_shape=jax.ShapeDtypeStruct((NPAD, D), jnp.float32),
    )(x_pad, W)


def _g_body(h_ref, d0_ref, d1_ref, g_ref):
    deg = (d0_ref[:, 0:1].astype(jnp.float32)
           + d1_ref[:, 0:1].astype(jnp.float32) + 1.0)
    g_ref[...] = (h_ref[...] * lax.rsqrt(deg)).astype(jnp.bfloat16)


def _tc_g(h, d0, d1):
    return pl.pallas_call(
        _g_body,
        grid=(NPAD // _BLK1,),
        in_specs=[
            pl.BlockSpec((_BLK1, D), lambda i: (i, 0)),
            pl.BlockSpec((_BLK1, D), lambda i: (i, 0)),
            pl.BlockSpec((_BLK1, D), lambda i: (i, 0)),
        ],
        out_specs=pl.BlockSpec((_BLK1, D), lambda i: (i, 0)),
        out_shape=jax.ShapeDtypeStruct((NPAD, D), jnp.bfloat16),
    )(h, d0, d1)


def _out_body(s0_ref, s1_ref, g_ref, d0_ref, d1_ref, b_ref, o_ref):
    deg = (d0_ref[:, 0:1].astype(jnp.float32)
           + d1_ref[:, 0:1].astype(jnp.float32) + 1.0)
    r = lax.rsqrt(deg)
    s = (s0_ref[...].astype(jnp.float32) + s1_ref[...].astype(jnp.float32)
         + g_ref[...].astype(jnp.float32))
    o_ref[...] = jnp.tanh(s * r + b_ref[...])


_BLK2 = 2000


def _tc_out(s0, s1, g, d0, d1, b2):
    return pl.pallas_call(
        _out_body,
        grid=(N // _BLK2,),
        in_specs=[
            pl.BlockSpec((_BLK2, D), lambda i: (i, 0)),
            pl.BlockSpec((_BLK2, D), lambda i: (i, 0)),
            pl.BlockSpec((_BLK2, D), lambda i: (i, 0)),
            pl.BlockSpec((_BLK2, D), lambda i: (i, 0)),
            pl.BlockSpec((_BLK2, D), lambda i: (i, 0)),
            pl.BlockSpec((1, D), lambda i: (0, 0)),
        ],
        out_specs=pl.BlockSpec((_BLK2, D), lambda i: (i, 0)),
        out_shape=jax.ShapeDtypeStruct((N, D), jnp.float32),
    )(s0, s1, g, d0, d1, b2)


def kernel(x, edge_index, W, b):
    src = edge_index[0].astype(jnp.int32)
    dst = edge_index[1].astype(jnp.int32)
    pad = jnp.full((EPAD - E,), N, dtype=jnp.int32)
    src3 = jnp.concatenate([src, pad]).reshape(NW, CH, CK)
    dst3 = jnp.concatenate([dst, pad]).reshape(NW, CH, CK)
    x_pad = jnp.pad(x, ((0, NPAD - N), (0, 0)))

    zerosS = jnp.zeros((NPAD, D), jnp.bfloat16)
    onesD = jnp.ones((CK, D), jnp.bfloat16)

    degpk = _sc_deg(dst3, zerosS, onesD)
    degb = lax.bitcast_convert_type(degpk, jnp.bfloat16).reshape(NC, NPAD, D)
    h = _tc_h(x_pad, W)
    g = _tc_g(h, degb[0], degb[1])
    gpk = lax.bitcast_convert_type(g.reshape(NPAD, HD, 2), jnp.float32)
    Spk = _sc_agg(gpk, src3, dst3, zerosS)
    Sb = lax.bitcast_convert_type(Spk, jnp.bfloat16).reshape(NC, NPAD, D)
    return _tc_out(Sb[0], Sb[1], g, degb[0], degb[1],
                   b.reshape(1, D).astype(jnp.float32))
